# baseline (speedup 1.0000x reference)
import jax
import jax.numpy as jnp
from jax.experimental import pallas as pl
from jax.experimental.pallas import tpu as pltpu

B, L, D = 16, 4096, 256
NQ = 4
TQ = L // NQ
G = TQ // 8
N = B * L
CACHE_SLOTS = 10


def _q_index_map(q):
    def imap(p, seq_ref, tab_ref, slot_ref, t2_ref):
        b = jnp.where(p < B, p, p - B)
        return (jnp.where(p < B, tab_ref[q, b], t2_ref[q, b]), q, 0)
    return imap


def _out_index_map(p, seq_ref, tab_ref, slot_ref, t2_ref):
    return (jnp.maximum(p - B, 0), 0, 0)


def _finalize(acc_ref, stats_ref):
    inv_n = 1.0 / N
    inv_nm1 = 1.0 / (N - 1)
    s_x = jnp.sum(acc_ref[0], axis=0, keepdims=True)
    ss_x = jnp.sum(acc_ref[1], axis=0, keepdims=True)
    s_y = jnp.sum(acc_ref[2], axis=0, keepdims=True)
    ss_y = jnp.sum(acc_ref[3], axis=0, keepdims=True)
    x_loc = s_x * inv_n
    y_loc = s_y * inv_n
    x_var = (ss_x - N * x_loc * x_loc) * inv_nm1
    y_var = (ss_y - N * y_loc * y_loc) * inv_nm1
    x_rs = jax.lax.rsqrt(x_var)
    y_rs = jax.lax.rsqrt(y_var)
    rows = jnp.concatenate([x_rs, -x_loc * x_rs, y_rs, -y_loc * y_rs], axis=0)
    stats_ref[...] = jnp.broadcast_to(rows[:, None, :], (4, 8, D))


def _fused_kernel(seq_ref, tab_ref, slot_ref, t2_ref, *refs):
    x_refs = refs[0:NQ]
    y_refs = refs[NQ:2 * NQ]
    xo_ref = refs[2 * NQ]
    yo_ref = refs[2 * NQ + 1]
    m_ref = refs[2 * NQ + 2]
    acc_ref = refs[2 * NQ + 3]
    stats_ref = refs[2 * NQ + 4]
    cx_ref = refs[2 * NQ + 5]
    cy_ref = refs[2 * NQ + 6]

    p = pl.program_id(0)

    @pl.when(p == 0)
    def _():
        acc_ref[...] = jnp.zeros_like(acc_ref)

    @pl.when(p < B)
    def _():
        slen = seq_ref[p]
        col0 = jax.lax.broadcasted_iota(jnp.int32, (8, D), 1) == 0

        def accumulate(xm, ym, d):
            acc_ref[0] += jnp.sum(xm, axis=0)
            acc_ref[1] += jnp.sum(xm * xm, axis=0)
            s_y = jnp.sum(ym, axis=0)
            ss_y = jnp.sum(ym * ym, axis=0)
            s_d = jnp.sum(d, axis=0)
            ss_d = jnp.sum(d * d, axis=0)
            acc_ref[2] += jnp.where(col0, s_d, s_y)
            acc_ref[3] += jnp.where(col0, ss_d, ss_y)

        for q in range(NQ):
            start = q * TQ
            slot = slot_ref[p, q]

            @pl.when(slot >= 0)
            def _(q=q, slot=slot):
                cx_ref[slot] = x_refs[q][0]
                cy_ref[slot] = y_refs[q][0]

            @pl.when(start + TQ <= slen)
            def _(q=q):
                xb = x_refs[q][0].reshape(G, 8, D)
                yb = y_refs[q][0].reshape(G, 8, D)
                d = xb[:, :, 0:1] - yb[:, :, 0:1]
                accumulate(xb, yb, d)

            @pl.when((start < slen) & (start + TQ > slen))
            def _(q=q, start=start):
                xb = x_refs[q][0].reshape(G, 8, D)
                yb = y_refs[q][0].reshape(G, 8, D)
                rows = (jax.lax.broadcasted_iota(jnp.int32, (G, 8, 1), 0) * 8
                        + jax.lax.broadcasted_iota(jnp.int32, (G, 8, 1), 1)
                        + start)
                valid = rows < slen
                xm = jnp.where(valid, xb, 0.0)
                ym = jnp.where(valid, yb, 0.0)
                d = xm[:, :, 0:1] - ym[:, :, 0:1]
                accumulate(xm, ym, d)

        @pl.when(p == B - 1)
        def _():
            _finalize(acc_ref, stats_ref)

    @pl.when(p >= B)
    def _():
        b = p - B
        slen = seq_ref[b]

        x_rs = stats_ref[0]
        x_fill = stats_ref[1]
        y_rs = stats_ref[2]
        y_fill = stats_ref[3]
        y_rs0 = stats_ref[2, :, 0:1]
        y_fill0 = stats_ref[3, :, 0:1]

        cols = jax.lax.broadcasted_iota(jnp.int32, (1, 1, L), 2)
        m_ref[...] = (cols < slen).astype(jnp.float32)

        def norm_full(xb, yb, sl):
            xo_ref[0, sl, :] = (xb * x_rs + x_fill).reshape(TQ, D)
            yo_ref[0, sl, :] = (yb * y_rs + y_fill).reshape(TQ, D)
            d = xb[:, :, 0:1] - yb[:, :, 0:1]
            yo_ref[0, sl, 0:1] = (d * y_rs0 + y_fill0).reshape(TQ, 1)

        def norm_boundary(xb, yb, sl, start):
            rows = (jax.lax.broadcasted_iota(jnp.int32, (G, 8, 1), 0) * 8
                    + jax.lax.broadcasted_iota(jnp.int32, (G, 8, 1), 1)
                    + start)
            valid = rows < slen
            xo_ref[0, sl, :] = jnp.where(
                valid, xb * x_rs + x_fill, x_fill).reshape(TQ, D)
            yo_ref[0, sl, :] = jnp.where(
                valid, yb * y_rs + y_fill, y_fill).reshape(TQ, D)
            d = xb[:, :, 0:1] - yb[:, :, 0:1]
            yo_ref[0, sl, 0:1] = jnp.where(
                valid, d * y_rs0 + y_fill0, y_fill0).reshape(TQ, 1)

        for q in range(NQ):
            start = q * TQ
            sl = slice(start, start + TQ)
            slot = slot_ref[b, q]
            cached = slot >= 0
            full = start + TQ <= slen
            boundary = (start < slen) & (start + TQ > slen)

            @pl.when(full & cached)
            def _(q=q, sl=sl, slot=slot):
                xb = cx_ref[slot].reshape(G, 8, D)
                yb = cy_ref[slot].reshape(G, 8, D)
                norm_full(xb, yb, sl)

            @pl.when(full & jnp.logical_not(cached))
            def _(q=q, sl=sl):
                xb = x_refs[q][0].reshape(G, 8, D)
                yb = y_refs[q][0].reshape(G, 8, D)
                norm_full(xb, yb, sl)

            @pl.when(boundary & cached)
            def _(q=q, sl=sl, start=start, slot=slot):
                xb = cx_ref[slot].reshape(G, 8, D)
                yb = cy_ref[slot].reshape(G, 8, D)
                norm_boundary(xb, yb, sl, start)

            @pl.when(boundary & jnp.logical_not(cached))
            def _(q=q, sl=sl, start=start):
                xb = x_refs[q][0].reshape(G, 8, D)
                yb = y_refs[q][0].reshape(G, 8, D)
                norm_boundary(xb, yb, sl, start)

            @pl.when(start >= slen)
            def _(sl=sl):
                xo_ref[0, sl, :] = jnp.broadcast_to(
                    x_fill, (G, 8, D)).reshape(TQ, D)
                yo_ref[0, sl, :] = jnp.broadcast_to(
                    y_fill, (G, 8, D)).reshape(TQ, D)


def kernel(x, y, seq_len):
    seq32 = seq_len.astype(jnp.int32)

    barange = jnp.arange(B, dtype=jnp.int32)
    thresh = (jnp.arange(NQ, dtype=jnp.int32) * TQ)[:, None]
    valid = seq32[None, :] > thresh

    idx = jnp.where(valid, barange[None, :], -1)
    tab = jnp.maximum(jax.lax.cummax(idx, axis=1), 0)

    valid_bq = valid.T
    cnt = jnp.cumsum(valid_bq.reshape(-1).astype(jnp.int32)) - 1
    slot = jnp.where(valid_bq.reshape(-1) & (cnt < CACHE_SLOTS),
                     cnt, -1).reshape(B, NQ).astype(jnp.int32)

    uncached = valid & (slot.T < 0)
    idx2 = jnp.where(uncached, barange[None, :], -1)
    t2 = jnp.maximum(jax.lax.cummax(idx2, axis=1), 0)

    q_in_specs = [pl.BlockSpec((1, TQ, D), _q_index_map(q)) for q in range(NQ)]

    x_out, y_out, mask_f = pl.pallas_call(
        _fused_kernel,
        grid_spec=pltpu.PrefetchScalarGridSpec(
            num_scalar_prefetch=4,
            grid=(2 * B,),
            in_specs=q_in_specs + q_in_specs,
            out_specs=[
                pl.BlockSpec((1, L, D), _out_index_map),
                pl.BlockSpec((1, L, D), _out_index_map),
                pl.BlockSpec((1, 1, L), _out_index_map),
            ],
            scratch_shapes=[pltpu.VMEM((4, 8, D), jnp.float32),
                            pltpu.VMEM((4, 8, D), jnp.float32),
                            pltpu.VMEM((CACHE_SLOTS, TQ, D), jnp.float32),
                            pltpu.VMEM((CACHE_SLOTS, TQ, D), jnp.float32)],
        ),
        out_shape=[
            jax.ShapeDtypeStruct((B, L, D), jnp.float32),
            jax.ShapeDtypeStruct((B, L, D), jnp.float32),
            jax.ShapeDtypeStruct((B, 1, L), jnp.float32),
        ],
        compiler_params=pltpu.CompilerParams(
            dimension_semantics=("arbitrary",)),
    )(seq32, tab, slot, t2, x, x, x, x, y, y, y, y)

    mask = mask_f.reshape(B, L).astype(bool)
    return (x_out, y_out, seq_len, mask)


# speedup vs baseline: 1.0986x; 1.0986x over previous
import jax
import jax.numpy as jnp
from jax.experimental import pallas as pl
from jax.experimental.pallas import tpu as pltpu

B, L, D = 16, 4096, 256
NQ = 4
TQ = L // NQ
G = TQ // 8
N = B * L
K = 8
KO = 2


def _finalize(acc_ref, stats_ref):
    inv_n = 1.0 / N
    inv_nm1 = 1.0 / (N - 1)
    s_x = jnp.sum(acc_ref[0], axis=0, keepdims=True)
    ss_x = jnp.sum(acc_ref[1], axis=0, keepdims=True)
    s_y = jnp.sum(acc_ref[2], axis=0, keepdims=True)
    ss_y = jnp.sum(acc_ref[3], axis=0, keepdims=True)
    x_loc = s_x * inv_n
    y_loc = s_y * inv_n
    x_var = (ss_x - N * x_loc * x_loc) * inv_nm1
    y_var = (ss_y - N * y_loc * y_loc) * inv_nm1
    x_rs = jax.lax.rsqrt(x_var)
    y_rs = jax.lax.rsqrt(y_var)
    rows = jnp.concatenate([x_rs, -x_loc * x_rs, y_rs, -y_loc * y_rs], axis=0)
    stats_ref[...] = jnp.broadcast_to(rows[:, None, :], (4, 8, D))


def _manual_kernel(seq_ref, work_ref, i0_ref, nv_ref,
                   x_ref, y_ref, xo_ref, yo_ref, m_ref,
                   bx_ref, by_ref, ox_ref, oy_ref, mbuf_ref,
                   acc_ref, stats_ref, rsem, wsem, msem):
    n = nv_ref[0]
    col0 = jax.lax.broadcasted_iota(jnp.int32, (8, D), 1) == 0

    def issue_read(i):
        wq = work_ref[i]
        b = wq // NQ
        q = wq % NQ
        slot = jax.lax.rem(i, jnp.int32(K))
        pltpu.make_async_copy(
            x_ref.at[b, pl.ds(q * TQ, TQ), :], bx_ref.at[slot],
            rsem.at[0, slot]).start()
        pltpu.make_async_copy(
            y_ref.at[b, pl.ds(q * TQ, TQ), :], by_ref.at[slot],
            rsem.at[1, slot]).start()

    def wait_read(slot):
        pltpu.make_async_copy(
            x_ref.at[0, pl.ds(0, TQ), :], bx_ref.at[slot],
            rsem.at[0, slot]).wait()
        pltpu.make_async_copy(
            y_ref.at[0, pl.ds(0, TQ), :], by_ref.at[slot],
            rsem.at[1, slot]).wait()

    def prologue():
        for k in range(K):
            @pl.when(k < n)
            def _(k=k):
                issue_read(jnp.int32(k))

    acc_ref[...] = jnp.zeros_like(acc_ref)
    prologue()

    def accumulate(xm, ym, d):
        acc_ref[0] += jnp.sum(xm, axis=0)
        acc_ref[1] += jnp.sum(xm * xm, axis=0)
        s_y = jnp.sum(ym, axis=0)
        ss_y = jnp.sum(ym * ym, axis=0)
        s_d = jnp.sum(d, axis=0)
        ss_d = jnp.sum(d * d, axis=0)
        acc_ref[2] += jnp.where(col0, s_d, s_y)
        acc_ref[3] += jnp.where(col0, ss_d, ss_y)

    def body0(i, _):
        wq = work_ref[i]
        b = wq // NQ
        q = wq % NQ
        slot = jax.lax.rem(i, jnp.int32(K))
        start = q * TQ
        slen = seq_ref[b]
        wait_read(slot)
        xb = bx_ref[slot].reshape(G, 8, D)
        yb = by_ref[slot].reshape(G, 8, D)
        full = start + TQ <= slen

        @pl.when(full)
        def _():
            d = xb[:, :, 0:1] - yb[:, :, 0:1]
            accumulate(xb, yb, d)

        @pl.when(jnp.logical_not(full))
        def _():
            rows = (jax.lax.broadcasted_iota(jnp.int32, (G, 8, 1), 0) * 8
                    + jax.lax.broadcasted_iota(jnp.int32, (G, 8, 1), 1)
                    + start)
            valid = rows < slen
            xm = jnp.where(valid, xb, 0.0)
            ym = jnp.where(valid, yb, 0.0)
            d = xm[:, :, 0:1] - ym[:, :, 0:1]
            accumulate(xm, ym, d)

        @pl.when(i + K < n)
        def _():
            issue_read(i + K)
        return 0

    jax.lax.fori_loop(0, n, body0, 0)
    _finalize(acc_ref, stats_ref)

    prologue()

    x_rs = stats_ref[0]
    x_fill = stats_ref[1]
    y_rs = stats_ref[2]
    y_fill = stats_ref[3]
    y_rs0 = stats_ref[2, :, 0:1]
    y_fill0 = stats_ref[3, :, 0:1]
    lcols = jax.lax.broadcasted_iota(jnp.int32, (1, L), 1)

    def wait_write(ob):
        pltpu.make_async_copy(
            ox_ref.at[ob], xo_ref.at[0], wsem.at[0, ob]).wait()
        pltpu.make_async_copy(
            oy_ref.at[ob], yo_ref.at[0], wsem.at[1, ob]).wait()

    def body1(b, _):
        slen = seq_ref[b]
        ob = jax.lax.rem(b, jnp.int32(KO))
        i0 = i0_ref[b]

        @pl.when(b >= KO)
        def _():
            wait_write(ob)

        mbuf_ref[pl.ds(b, 1), :] = (lcols < slen).astype(jnp.float32)

        for q in range(NQ):
            start = q * TQ
            sl = slice(start, start + TQ)
            j = i0 + q
            slot = jax.lax.rem(j, jnp.int32(K))
            full = start + TQ <= slen
            boundary = (start < slen) & (start + TQ > slen)

            @pl.when(full)
            def _(sl=sl, slot=slot):
                wait_read(slot)
                xb = bx_ref[slot].reshape(G, 8, D)
                yb = by_ref[slot].reshape(G, 8, D)
                ox_ref[ob, sl, :] = (xb * x_rs + x_fill).reshape(TQ, D)
                oy_ref[ob, sl, :] = (yb * y_rs + y_fill).reshape(TQ, D)
                d = xb[:, :, 0:1] - yb[:, :, 0:1]
                oy_ref[ob, sl, 0:1] = (d * y_rs0 + y_fill0).reshape(TQ, 1)

            @pl.when(boundary)
            def _(sl=sl, slot=slot, start=start):
                wait_read(slot)
                xb = bx_ref[slot].reshape(G, 8, D)
                yb = by_ref[slot].reshape(G, 8, D)
                rows = (jax.lax.broadcasted_iota(jnp.int32, (G, 8, 1), 0) * 8
                        + jax.lax.broadcasted_iota(jnp.int32, (G, 8, 1), 1)
                        + start)
                valid = rows < slen
                ox_ref[ob, sl, :] = jnp.where(
                    valid, xb * x_rs + x_fill, x_fill).reshape(TQ, D)
                oy_ref[ob, sl, :] = jnp.where(
                    valid, yb * y_rs + y_fill, y_fill).reshape(TQ, D)
                d = xb[:, :, 0:1] - yb[:, :, 0:1]
                oy_ref[ob, sl, 0:1] = jnp.where(
                    valid, d * y_rs0 + y_fill0, y_fill0).reshape(TQ, 1)

            @pl.when(start >= slen)
            def _(sl=sl):
                ox_ref[ob, sl, :] = jnp.broadcast_to(
                    x_fill, (G, 8, D)).reshape(TQ, D)
                oy_ref[ob, sl, :] = jnp.broadcast_to(
                    y_fill, (G, 8, D)).reshape(TQ, D)

            @pl.when((start < slen) & (j + K < n))
            def _(j=j):
                issue_read(j + K)

        pltpu.make_async_copy(
            ox_ref.at[ob], xo_ref.at[b], wsem.at[0, ob]).start()
        pltpu.make_async_copy(
            oy_ref.at[ob], yo_ref.at[b], wsem.at[1, ob]).start()
        return 0

    jax.lax.fori_loop(0, B, body1, 0)

    for k in range(KO):
        wait_write(jnp.int32((B - KO + k) % KO))

    mcopy = pltpu.make_async_copy(mbuf_ref, m_ref, msem)
    mcopy.start()
    mcopy.wait()


def kernel(x, y, seq_len):
    seq32 = seq_len.astype(jnp.int32)

    nq = (seq32 + (TQ - 1)) // TQ
    i0 = jnp.concatenate([jnp.zeros((1,), jnp.int32),
                          jnp.cumsum(nq)[:-1].astype(jnp.int32)])
    nv = jnp.sum(nq).astype(jnp.int32).reshape(1)
    bq = jnp.arange(B * NQ, dtype=jnp.int32)
    bb, qq = bq // NQ, bq % NQ
    is_item = qq < nq[bb]
    pos = jnp.where(is_item, i0[bb] + qq, B * NQ)
    work = jnp.zeros((B * NQ + 1,), jnp.int32).at[pos].set(bq)[:B * NQ]

    x_out, y_out, mask_f = pl.pallas_call(
        _manual_kernel,
        in_specs=[
            pl.BlockSpec(memory_space=pltpu.SMEM),
            pl.BlockSpec(memory_space=pltpu.SMEM),
            pl.BlockSpec(memory_space=pltpu.SMEM),
            pl.BlockSpec(memory_space=pltpu.SMEM),
            pl.BlockSpec(memory_space=pl.ANY),
            pl.BlockSpec(memory_space=pl.ANY),
        ],
        out_specs=[
            pl.BlockSpec(memory_space=pl.ANY),
            pl.BlockSpec(memory_space=pl.ANY),
            pl.BlockSpec(memory_space=pl.ANY),
        ],
        out_shape=[
            jax.ShapeDtypeStruct((B, L, D), jnp.float32),
            jax.ShapeDtypeStruct((B, L, D), jnp.float32),
            jax.ShapeDtypeStruct((B, L), jnp.float32),
        ],
        scratch_shapes=[
            pltpu.VMEM((K, TQ, D), jnp.float32),
            pltpu.VMEM((K, TQ, D), jnp.float32),
            pltpu.VMEM((KO, L, D), jnp.float32),
            pltpu.VMEM((KO, L, D), jnp.float32),
            pltpu.VMEM((B, L), jnp.float32),
            pltpu.VMEM((4, 8, D), jnp.float32),
            pltpu.VMEM((4, 8, D), jnp.float32),
            pltpu.SemaphoreType.DMA((2, K)),
            pltpu.SemaphoreType.DMA((2, KO)),
            pltpu.SemaphoreType.DMA,
        ],
    )(seq32, work, i0, nv, x, y)

    mask = mask_f.astype(bool)
    return (x_out, y_out, seq_len, mask)


# speedup vs baseline: 1.0989x; 1.0003x over previous
import jax
import jax.numpy as jnp
from jax.experimental import pallas as pl
from jax.experimental.pallas import tpu as pltpu

B, L, D = 16, 4096, 256
NQ = 4
TQ = L // NQ
G = TQ // 8
N = B * L
K = 12
KO = 2


def _finalize(acc_ref, stats_ref):
    inv_n = 1.0 / N
    inv_nm1 = 1.0 / (N - 1)
    s_x = jnp.sum(acc_ref[0], axis=0, keepdims=True)
    ss_x = jnp.sum(acc_ref[1], axis=0, keepdims=True)
    s_y = jnp.sum(acc_ref[2], axis=0, keepdims=True)
    ss_y = jnp.sum(acc_ref[3], axis=0, keepdims=True)
    x_loc = s_x * inv_n
    y_loc = s_y * inv_n
    x_var = (ss_x - N * x_loc * x_loc) * inv_nm1
    y_var = (ss_y - N * y_loc * y_loc) * inv_nm1
    x_rs = jax.lax.rsqrt(x_var)
    y_rs = jax.lax.rsqrt(y_var)
    rows = jnp.concatenate([x_rs, -x_loc * x_rs, y_rs, -y_loc * y_rs], axis=0)
    stats_ref[...] = jnp.broadcast_to(rows[:, None, :], (4, 8, D))


def _manual_kernel(seq_ref, work_ref, i0_ref, nv_ref,
                   x_ref, y_ref, xo_ref, yo_ref, m_ref,
                   bx_ref, by_ref, ox_ref, oy_ref, mbuf_ref,
                   acc_ref, stats_ref, rsem, wsem, msem):
    n = nv_ref[0]
    col0 = jax.lax.broadcasted_iota(jnp.int32, (8, D), 1) == 0

    def issue_read(i):
        wq = work_ref[i]
        b = wq // NQ
        q = wq % NQ
        slot = jax.lax.rem(i, jnp.int32(K))
        pltpu.make_async_copy(
            x_ref.at[b, pl.ds(q * TQ, TQ), :], bx_ref.at[slot],
            rsem.at[0, slot]).start()
        pltpu.make_async_copy(
            y_ref.at[b, pl.ds(q * TQ, TQ), :], by_ref.at[slot],
            rsem.at[1, slot]).start()

    def wait_read(slot):
        pltpu.make_async_copy(
            x_ref.at[0, pl.ds(0, TQ), :], bx_ref.at[slot],
            rsem.at[0, slot]).wait()
        pltpu.make_async_copy(
            y_ref.at[0, pl.ds(0, TQ), :], by_ref.at[slot],
            rsem.at[1, slot]).wait()

    def prologue():
        for k in range(K):
            @pl.when(k < n)
            def _(k=k):
                issue_read(jnp.int32(k))

    acc_ref[...] = jnp.zeros_like(acc_ref)
    prologue()

    def accumulate(xm, ym, d):
        acc_ref[0] += jnp.sum(xm, axis=0)
        acc_ref[1] += jnp.sum(xm * xm, axis=0)
        s_y = jnp.sum(ym, axis=0)
        ss_y = jnp.sum(ym * ym, axis=0)
        s_d = jnp.sum(d, axis=0)
        ss_d = jnp.sum(d * d, axis=0)
        acc_ref[2] += jnp.where(col0, s_d, s_y)
        acc_ref[3] += jnp.where(col0, ss_d, ss_y)

    def body0(i, _):
        wq = work_ref[i]
        b = wq // NQ
        q = wq % NQ
        slot = jax.lax.rem(i, jnp.int32(K))
        start = q * TQ
        slen = seq_ref[b]
        wait_read(slot)
        xb = bx_ref[slot].reshape(G, 8, D)
        yb = by_ref[slot].reshape(G, 8, D)
        full = start + TQ <= slen

        @pl.when(full)
        def _():
            d = xb[:, :, 0:1] - yb[:, :, 0:1]
            accumulate(xb, yb, d)

        @pl.when(jnp.logical_not(full))
        def _():
            rows = (jax.lax.broadcasted_iota(jnp.int32, (G, 8, 1), 0) * 8
                    + jax.lax.broadcasted_iota(jnp.int32, (G, 8, 1), 1)
                    + start)
            valid = rows < slen
            xm = jnp.where(valid, xb, 0.0)
            ym = jnp.where(valid, yb, 0.0)
            d = xm[:, :, 0:1] - ym[:, :, 0:1]
            accumulate(xm, ym, d)

        @pl.when(i + K < n)
        def _():
            issue_read(i + K)
        return 0

    jax.lax.fori_loop(0, n, body0, 0)
    _finalize(acc_ref, stats_ref)

    prologue()

    x_rs = stats_ref[0]
    x_fill = stats_ref[1]
    y_rs = stats_ref[2]
    y_fill = stats_ref[3]
    y_rs0 = stats_ref[2, :, 0:1]
    y_fill0 = stats_ref[3, :, 0:1]
    lcols = jax.lax.broadcasted_iota(jnp.int32, (1, L), 1)

    def wait_write(ob):
        pltpu.make_async_copy(
            ox_ref.at[ob], xo_ref.at[0], wsem.at[0, ob]).wait()
        pltpu.make_async_copy(
            oy_ref.at[ob], yo_ref.at[0], wsem.at[1, ob]).wait()

    def body1(b, _):
        slen = seq_ref[b]
        ob = jax.lax.rem(b, jnp.int32(KO))
        i0 = i0_ref[b]

        @pl.when(b >= KO)
        def _():
            wait_write(ob)

        mbuf_ref[pl.ds(b, 1), :] = (lcols < slen).astype(jnp.float32)

        for q in range(NQ):
            start = q * TQ
            sl = slice(start, start + TQ)
            j = i0 + q
            slot = jax.lax.rem(j, jnp.int32(K))
            full = start + TQ <= slen
            boundary = (start < slen) & (start + TQ > slen)

            @pl.when(full)
            def _(sl=sl, slot=slot):
                wait_read(slot)
                xb = bx_ref[slot].reshape(G, 8, D)
                yb = by_ref[slot].reshape(G, 8, D)
                ox_ref[ob, sl, :] = (xb * x_rs + x_fill).reshape(TQ, D)
                oy_ref[ob, sl, :] = (yb * y_rs + y_fill).reshape(TQ, D)
                d = xb[:, :, 0:1] - yb[:, :, 0:1]
                oy_ref[ob, sl, 0:1] = (d * y_rs0 + y_fill0).reshape(TQ, 1)

            @pl.when(boundary)
            def _(sl=sl, slot=slot, start=start):
                wait_read(slot)
                xb = bx_ref[slot].reshape(G, 8, D)
                yb = by_ref[slot].reshape(G, 8, D)
                rows = (jax.lax.broadcasted_iota(jnp.int32, (G, 8, 1), 0) * 8
                        + jax.lax.broadcasted_iota(jnp.int32, (G, 8, 1), 1)
                        + start)
                valid = rows < slen
                ox_ref[ob, sl, :] = jnp.where(
                    valid, xb * x_rs + x_fill, x_fill).reshape(TQ, D)
                oy_ref[ob, sl, :] = jnp.where(
                    valid, yb * y_rs + y_fill, y_fill).reshape(TQ, D)
                d = xb[:, :, 0:1] - yb[:, :, 0:1]
                oy_ref[ob, sl, 0:1] = jnp.where(
                    valid, d * y_rs0 + y_fill0, y_fill0).reshape(TQ, 1)

            @pl.when(start >= slen)
            def _(sl=sl):
                ox_ref[ob, sl, :] = jnp.broadcast_to(
                    x_fill, (G, 8, D)).reshape(TQ, D)
                oy_ref[ob, sl, :] = jnp.broadcast_to(
                    y_fill, (G, 8, D)).reshape(TQ, D)

            @pl.when((start < slen) & (j + K < n))
            def _(j=j):
                issue_read(j + K)

        pltpu.make_async_copy(
            ox_ref.at[ob], xo_ref.at[b], wsem.at[0, ob]).start()
        pltpu.make_async_copy(
            oy_ref.at[ob], yo_ref.at[b], wsem.at[1, ob]).start()
        return 0

    jax.lax.fori_loop(0, B, body1, 0)

    for k in range(KO):
        wait_write(jnp.int32((B - KO + k) % KO))

    mcopy = pltpu.make_async_copy(mbuf_ref, m_ref, msem)
    mcopy.start()
    mcopy.wait()


def kernel(x, y, seq_len):
    seq32 = seq_len.astype(jnp.int32)

    nq = (seq32 + (TQ - 1)) // TQ
    i0 = jnp.concatenate([jnp.zeros((1,), jnp.int32),
                          jnp.cumsum(nq)[:-1].astype(jnp.int32)])
    nv = jnp.sum(nq).astype(jnp.int32).reshape(1)
    bq = jnp.arange(B * NQ, dtype=jnp.int32)
    bb, qq = bq // NQ, bq % NQ
    is_item = qq < nq[bb]
    pos = jnp.where(is_item, i0[bb] + qq, B * NQ)
    work = jnp.zeros((B * NQ + 1,), jnp.int32).at[pos].set(bq)[:B * NQ]

    x_out, y_out, mask_f = pl.pallas_call(
        _manual_kernel,
        in_specs=[
            pl.BlockSpec(memory_space=pltpu.SMEM),
            pl.BlockSpec(memory_space=pltpu.SMEM),
            pl.BlockSpec(memory_space=pltpu.SMEM),
            pl.BlockSpec(memory_space=pltpu.SMEM),
            pl.BlockSpec(memory_space=pl.ANY),
            pl.BlockSpec(memory_space=pl.ANY),
        ],
        out_specs=[
            pl.BlockSpec(memory_space=pl.ANY),
            pl.BlockSpec(memory_space=pl.ANY),
            pl.BlockSpec(memory_space=pl.ANY),
        ],
        out_shape=[
            jax.ShapeDtypeStruct((B, L, D), jnp.float32),
            jax.ShapeDtypeStruct((B, L, D), jnp.float32),
            jax.ShapeDtypeStruct((B, L), jnp.float32),
        ],
        scratch_shapes=[
            pltpu.VMEM((K, TQ, D), jnp.float32),
            pltpu.VMEM((K, TQ, D), jnp.float32),
            pltpu.VMEM((KO, L, D), jnp.float32),
            pltpu.VMEM((KO, L, D), jnp.float32),
            pltpu.VMEM((B, L), jnp.float32),
            pltpu.VMEM((4, 8, D), jnp.float32),
            pltpu.VMEM((4, 8, D), jnp.float32),
            pltpu.SemaphoreType.DMA((2, K)),
            pltpu.SemaphoreType.DMA((2, KO)),
            pltpu.SemaphoreType.DMA,
        ],
    )(seq32, work, i0, nv, x, y)

    mask = mask_f.astype(bool)
    return (x_out, y_out, seq_len, mask)
